# Initial kernel scaffold; baseline (speedup 1.0000x reference)
import jax
import jax.numpy as jnp
from jax.experimental import pallas as pl


def kernel(hidden_states, router_weight, w1, w2):
    raise NotImplementedError("write your pallas kernel here")


# speedup vs baseline: 1.2288x; 1.2288x over previous
import functools

import jax
import jax.numpy as jnp
from jax.experimental import pallas as pl
from jax.experimental.pallas import tpu as pltpu

HIDDEN = 2048
NUM_EXPERTS = 8
FF = 768
BM = 256


def _router_body(x_ref, rw_ref, logits_ref, merge_ref):
    x = x_ref[...]
    rw = rw_ref[...]
    logits = jax.lax.dot_general(
        x, rw, (((1,), (1,)), ((), ())),
        preferred_element_type=jnp.float32)
    logits_ref[...] = logits
    m = jnp.max(logits, axis=1, keepdims=True)
    ex = jnp.exp(logits - m)
    probs = ex / jnp.sum(ex, axis=1, keepdims=True)
    iota = jax.lax.broadcasted_iota(jnp.int32, probs.shape, 1)
    m1 = jnp.max(probs, axis=1, keepdims=True)
    idx1 = jnp.min(jnp.where(probs == m1, iota, NUM_EXPERTS), axis=1,
                   keepdims=True)
    probs_m = jnp.where(iota == idx1, -1.0, probs)
    m2 = jnp.max(probs_m, axis=1, keepdims=True)
    idx2 = jnp.min(jnp.where(probs_m == m2, iota, NUM_EXPERTS), axis=1,
                   keepdims=True)
    s = m1 + m2
    merge_ref[...] = (jnp.where(iota == idx1, m1 / s, 0.0)
                      + jnp.where(iota == idx2, m2 / s, 0.0))


def _expert_body(x_ref, w1_ref, w2_ref, mp_ref, out_ref, acc_ref):
    e = pl.program_id(0)
    t = pl.program_id(1)
    x = x_ref[...]
    h = jax.lax.dot_general(
        x, w1_ref[0], (((1,), (1,)), ((), ())),
        preferred_element_type=jnp.float32)
    gate = h[:, :FF]
    up = h[:, FF:]
    act = (gate * jax.nn.sigmoid(gate) * up).astype(jnp.bfloat16)
    y = jax.lax.dot_general(
        act, w2_ref[0], (((1,), (1,)), ((), ())),
        preferred_element_type=jnp.float32)
    mp = mp_ref[...]
    lane = jax.lax.broadcasted_iota(jnp.int32, mp.shape, 1)
    w = jnp.sum(jnp.where(lane == e, mp, 0.0), axis=1, keepdims=True)
    contrib = w * y
    sl = pl.ds(t * BM, BM)

    @pl.when(e == 0)
    def _():
        acc_ref[sl, :] = contrib

    @pl.when(jnp.logical_and(e > 0, e < NUM_EXPERTS - 1))
    def _():
        acc_ref[sl, :] = acc_ref[sl, :] + contrib

    @pl.when(e == NUM_EXPERTS - 1)
    def _():
        out_ref[...] = acc_ref[sl, :] + contrib


@jax.jit
def kernel(hidden_states, router_weight, w1, w2):
    b, s, d = hidden_states.shape
    flat = hidden_states.reshape(-1, d)
    T = flat.shape[0]
    tb = T // BM

    logits, merge = pl.pallas_call(
        _router_body,
        grid=(tb,),
        in_specs=[
            pl.BlockSpec((BM, d), lambda t: (t, 0)),
            pl.BlockSpec((NUM_EXPERTS, d), lambda t: (0, 0)),
        ],
        out_specs=[
            pl.BlockSpec((BM, NUM_EXPERTS), lambda t: (t, 0)),
            pl.BlockSpec((BM, NUM_EXPERTS), lambda t: (t, 0)),
        ],
        out_shape=[
            jax.ShapeDtypeStruct((T, NUM_EXPERTS), jnp.float32),
            jax.ShapeDtypeStruct((T, NUM_EXPERTS), jnp.float32),
        ],
    )(flat, router_weight)

    xb = flat.astype(jnp.bfloat16)
    w1b = w1.astype(jnp.bfloat16)
    w2b = w2.astype(jnp.bfloat16)

    out = pl.pallas_call(
        _expert_body,
        grid=(NUM_EXPERTS, tb),
        in_specs=[
            pl.BlockSpec((BM, d), lambda e, t: (t, 0)),
            pl.BlockSpec((1, 2 * FF, d), lambda e, t: (e, 0, 0)),
            pl.BlockSpec((1, d, FF), lambda e, t: (e, 0, 0)),
            pl.BlockSpec((BM, NUM_EXPERTS), lambda e, t: (t, 0)),
        ],
        out_specs=pl.BlockSpec(
            (BM, d), lambda e, t: (jnp.where(e == NUM_EXPERTS - 1, t, 0), 0)),
        out_shape=jax.ShapeDtypeStruct((T, d), jnp.float32),
        scratch_shapes=[pltpu.VMEM((T, d), jnp.float32)],
    )(xb, w1b, w2b, merge)

    return out.reshape(b, s, d), logits
